# baseline (speedup 1.0000x reference)
import jax
import jax.numpy as jnp
from jax import lax
from jax.experimental import pallas as pl
from jax.experimental.pallas import tpu as pltpu
from jax.experimental.pallas import tpu_sc as plsc

DIM = 512
NW = 32
CHUNK = 32
NBUF = 7


def _gather_body(feats_hbm, table_hbm, out_hbm, idx_all,
                 buf0, buf1, buf2, buf3, buf4, buf5, buf6,
                 gs0, gs1, gs2, gs3, gs4, gs5, gs6,
                 ss0, ss1, ss2, ss3, ss4, ss5, ss6):
    wid = lax.axis_index("s") * 2 + lax.axis_index("c")
    n = feats_hbm.shape[0]
    per_w = n // NW
    chunks = per_w // CHUNK
    start = wid * per_w

    bufs = (buf0, buf1, buf2, buf3, buf4, buf5, buf6)
    gs = (gs0, gs1, gs2, gs3, gs4, gs5, gs6)
    ss = (ss0, ss1, ss2, ss3, ss4, ss5, ss6)

    def out_slc(i):
        return out_hbm.at[pl.ds(start + i * CHUNK, CHUNK)]

    def idx_slc(i):
        return idx_all.at[pl.ds(i * CHUNK, CHUNK)]

    def gather(i, b):
        pltpu.async_copy(table_hbm.at[idx_slc(i)], bufs[b], gs[b])

    def gather_wait(i, b):
        pltpu.make_async_copy(table_hbm.at[idx_slc(i)], bufs[b], gs[b]).wait()

    def store(i, b):
        pltpu.async_copy(bufs[b], out_slc(i), ss[b])

    def store_wait(i, b):
        pltpu.make_async_copy(bufs[b], out_slc(i), ss[b]).wait()

    pltpu.sync_copy(feats_hbm.at[pl.ds(start, per_w)], idx_all)

    gather(0, 0)
    gather(1, 1)
    gather(2, 2)

    for k in range(0, 4):
        gather_wait(k, k % NBUF)
        store(k, k % NBUF)
        gather(k + 3, (k + 3) % NBUF)

    def body(k, b, b2):
        store_wait(k - 4, b2)
        gather(k + 3, b2)
        gather_wait(k, b)
        store(k, b)

    def step(j, carry):
        k = NBUF * j + 4
        for t in range(NBUF):
            body(k + t, (4 + t) % NBUF, t % NBUF)
        return carry

    q = (chunks - 7) // NBUF
    lax.fori_loop(0, q, step, 0)
    for k in range(4 + NBUF * q, chunks - 3):
        body(k, k % NBUF, (k + 3) % NBUF)

    for k in range(chunks - 3, chunks):
        gather_wait(k, k % NBUF)
        store(k, k % NBUF)
        store_wait(k - 4, (k - 4) % NBUF)
    for k in range(chunks - 4, chunks):
        store_wait(k, k % NBUF)


def kernel(feats, table):
    B, T = feats.shape
    flat = feats.reshape(B * T)
    per_w = (B * T) // NW
    mesh = plsc.VectorSubcoreMesh(core_axis_name="c", subcore_axis_name="s")
    out = pl.kernel(
        _gather_body,
        mesh=mesh,
        out_type=jax.ShapeDtypeStruct((B * T, DIM), jnp.float32),
        scratch_types=(
            [pltpu.VMEM((per_w,), jnp.int32)]
            + [pltpu.VMEM((CHUNK, DIM), jnp.float32)] * NBUF
            + [pltpu.SemaphoreType.DMA] * (2 * NBUF)
        ),
    )(flat, table)
    return out.reshape(B, T, DIM)


# speedup vs baseline: 1.0371x; 1.0371x over previous
import jax
import jax.numpy as jnp
from jax import lax
from jax.experimental import pallas as pl
from jax.experimental.pallas import tpu as pltpu
from jax.experimental.pallas import tpu_sc as plsc

DIM = 512
NW = 32
CHUNK = 32
NBUF = 4
NSLOT = 3
PERIOD = 12


def _gather_body(feats_hbm, table_hbm, out_hbm, idx_all,
                 buf0, buf1, buf2, buf3, spm,
                 gs0, gs1, gs2, gs3,
                 xs0, xs1, xs2, xs3,
                 hs0, hs1, hs2):
    cid = lax.axis_index("c")
    sid = lax.axis_index("s")
    wid = sid * 2 + cid
    n = feats_hbm.shape[0]
    per_w = n // NW
    chunks = per_w // CHUNK
    start = wid * per_w

    bufs = (buf0, buf1, buf2, buf3)
    gs = (gs0, gs1, gs2, gs3)
    xs = (xs0, xs1, xs2, xs3)
    hs = (hs0, hs1, hs2)

    def out_slc(i):
        return out_hbm.at[pl.ds(start + i * CHUNK, CHUNK)]

    def idx_slc(i):
        return idx_all.at[pl.ds(i * CHUNK, CHUNK)]

    def slot(s):
        return spm.at[pl.ds((sid * NSLOT + s) * CHUNK, CHUNK)]

    def gather(i, b):
        pltpu.async_copy(table_hbm.at[idx_slc(i)], bufs[b], gs[b])

    def gather_wait(i, b):
        pltpu.make_async_copy(table_hbm.at[idx_slc(i)], bufs[b], gs[b]).wait()

    def xfer(b, s):
        pltpu.async_copy(bufs[b], slot(s), xs[b])

    def xfer_wait(b, s):
        pltpu.make_async_copy(bufs[b], slot(s), xs[b]).wait()

    def hstore(i, s):
        pltpu.async_copy(slot(s), out_slc(i), hs[s])

    def hstore_wait(i, s):
        pltpu.make_async_copy(slot(s), out_slc(i), hs[s]).wait()

    pltpu.sync_copy(feats_hbm.at[pl.ds(start, per_w)], idx_all)

    gather(0, 0)
    gather(1, 1)
    gather(2, 2)

    gather_wait(0, 0)
    xfer(0, 0)
    gather(3, 3)
    gather_wait(1, 1)
    xfer(1, 1)
    xfer_wait(0, 0)
    hstore(0, 0)
    gather(4, 0)
    gather_wait(2, 2)
    xfer(2, 2)
    xfer_wait(1, 1)
    hstore(1, 1)
    gather(5, 1)

    def body(k, p):
        b = p % NBUF
        s = p % NSLOT
        gather_wait(k, b)
        hstore_wait(k - NSLOT, s)
        xfer(b, s)
        xfer_wait((p - 1) % NBUF, (p - 1) % NSLOT)
        hstore(k - 1, (p - 1) % NSLOT)
        gather(k + 3, (p + 3) % NBUF)

    def step(j, carry):
        k = PERIOD * j + 3
        for t in range(PERIOD):
            body(k + t, (3 + t) % PERIOD)
        return carry

    q = (chunks - 6) // PERIOD
    lax.fori_loop(0, q, step, 0)
    for k in range(3 + PERIOD * q, chunks - 3):
        body(k, k % PERIOD)

    for k in range(chunks - 3, chunks):
        gather_wait(k, k % NBUF)
        hstore_wait(k - NSLOT, k % NSLOT)
        xfer(k % NBUF, k % NSLOT)
        xfer_wait((k - 1) % NBUF, (k - 1) % NSLOT)
        hstore(k - 1, (k - 1) % NSLOT)
    xfer_wait((chunks - 1) % NBUF, (chunks - 1) % NSLOT)
    hstore(chunks - 1, (chunks - 1) % NSLOT)
    for k in range(chunks - NSLOT, chunks):
        hstore_wait(k, k % NSLOT)


def kernel(feats, table):
    B, T = feats.shape
    flat = feats.reshape(B * T)
    per_w = (B * T) // NW
    mesh = plsc.VectorSubcoreMesh(core_axis_name="c", subcore_axis_name="s")
    out = pl.kernel(
        _gather_body,
        mesh=mesh,
        out_type=jax.ShapeDtypeStruct((B * T, DIM), jnp.float32),
        scratch_types=(
            [pltpu.VMEM((per_w,), jnp.int32)]
            + [pltpu.VMEM((CHUNK, DIM), jnp.float32)] * NBUF
            + [pltpu.VMEM_SHARED((16 * NSLOT * CHUNK, DIM), jnp.float32)]
            + [pltpu.SemaphoreType.DMA] * (NBUF + NBUF + NSLOT)
        ),
    )(flat, table)
    return out.reshape(B, T, DIM)
